# Initial kernel scaffold; baseline (speedup 1.0000x reference)
import jax
import jax.numpy as jnp
from jax.experimental import pallas as pl


def kernel(x, batch, weight, bias):
    raise NotImplementedError("write your pallas kernel here")


# speedup vs baseline: 9.6642x; 9.6642x over previous
import functools

import jax
import jax.numpy as jnp
from jax import lax
from jax.experimental import pallas as pl
from jax.experimental.pallas import tpu as pltpu
from jax.experimental.pallas import tpu_sc as plsc

N = 100000
C = 480
G = 64
CS = 128
EPS = 1e-05
CH = 40
NCH = N // CH
NC, NS = 2, 16
NW = NC * NS
L = 16

_f32 = jnp.float32


@functools.partial(
    pl.kernel,
    out_type=[
        jax.ShapeDtypeStruct((NW, G, C), _f32),
        jax.ShapeDtypeStruct((NW, G, CS), _f32),
        jax.ShapeDtypeStruct((NW, G, L), _f32),
    ],
    mesh=plsc.VectorSubcoreMesh(core_axis_name="c", subcore_axis_name="s"),
    scratch_types=[
        pltpu.VMEM((CH, C), _f32),
        pltpu.VMEM((CH + L,), jnp.int32),
        pltpu.VMEM((G, C), _f32),
        pltpu.VMEM((G, CS), _f32),
        pltpu.VMEM((G, L), _f32),
    ],
)
def _stats_kernel(x_hbm, batch_hbm, sums_hbm, sq_hbm, cnt_hbm,
                  xbuf, idx, acc, acc2, cnt):
    cid = lax.axis_index("c")
    sid = lax.axis_index("s")
    wid = cid * NS + sid

    zero = jnp.zeros((L,), _f32)

    def _zero(g, carry):
        for j in range(C // L):
            acc[g, pl.ds(j * L, L)] = zero
        for j in range(CS // L):
            acc2[g, pl.ds(j * L, L)] = zero
        cnt[g, pl.ds(0, L)] = zero
        return carry
    lax.fori_loop(0, G, _zero, 0)

    ntrips = (NCH - wid + NW - 1) // NW
    one = jnp.ones((L,), _f32)

    def _chunk(t, carry):
        base = (wid + t * NW) * CH
        pltpu.sync_copy(x_hbm.at[pl.ds(base, CH)], xbuf)
        pltpu.sync_copy(batch_hbm.at[pl.ds(base, CH)], idx.at[pl.ds(0, CH)])

        def _row(r, c2):
            g = idx[pl.ds(r, L)][0]
            for j in range(C // L):
                v = xbuf[r, pl.ds(j * L, L)]
                plsc.addupdate(acc.at[g, pl.ds(j * L, L)], v)
                if j < CS // L:
                    plsc.addupdate(acc2.at[g, pl.ds(j * L, L)], v * v)
            plsc.addupdate(cnt.at[g, pl.ds(0, L)], one)
            return c2
        lax.fori_loop(0, CH, _row, 0)
        return carry

    lax.fori_loop(0, ntrips, _chunk, 0)

    pltpu.sync_copy(acc, sums_hbm.at[wid])
    pltpu.sync_copy(acc2, sq_hbm.at[wid])
    pltpu.sync_copy(cnt, cnt_hbm.at[wid])


@functools.partial(
    pl.kernel,
    out_type=jax.ShapeDtypeStruct((N, C), _f32),
    mesh=plsc.VectorSubcoreMesh(core_axis_name="c", subcore_axis_name="s"),
    scratch_types=[
        pltpu.VMEM((CH, C), _f32),
        pltpu.VMEM((CH + L,), jnp.int32),
        pltpu.VMEM((G, C), _f32),
        pltpu.VMEM((G, C), _f32),
    ],
)
def _apply_kernel(x_hbm, batch_hbm, alpha_hbm, beta_hbm, out_hbm,
                  xbuf, idx, al, be):
    cid = lax.axis_index("c")
    sid = lax.axis_index("s")
    wid = cid * NS + sid

    pltpu.sync_copy(alpha_hbm, al)
    pltpu.sync_copy(beta_hbm, be)

    ntrips = (NCH - wid + NW - 1) // NW

    def _chunk(t, carry):
        base = (wid + t * NW) * CH
        pltpu.sync_copy(x_hbm.at[pl.ds(base, CH)], xbuf)
        pltpu.sync_copy(batch_hbm.at[pl.ds(base, CH)], idx.at[pl.ds(0, CH)])

        def _row(r, c2):
            g = idx[pl.ds(r, L)][0]
            for j in range(C // L):
                sl = pl.ds(j * L, L)
                xbuf[r, sl] = xbuf[r, sl] * al[g, sl] + be[g, sl]
            return c2
        lax.fori_loop(0, CH, _row, 0)

        pltpu.sync_copy(xbuf, out_hbm.at[pl.ds(base, CH)])
        return carry

    lax.fori_loop(0, ntrips, _chunk, 0)


def kernel(x, batch, weight, bias):
    batch = batch.astype(jnp.int32)
    sums_p, sq_p, cnt_p = _stats_kernel(x, batch)

    S = sums_p.sum(axis=0)
    Q = sq_p.sum(axis=(0, 2))
    cnt = cnt_p.sum(axis=0)[:, 0]
    cntc = jnp.maximum(cnt, 1.0)

    m = S[:, :CS].sum(axis=1) / (CS * cntc)
    v = jnp.maximum(Q / (CS * cntc) - m * m, 0.0)
    inv = 1.0 / (jnp.sqrt(v) + EPS)
    muT = S[:, 128:320].reshape(G, 64, 3).sum(axis=1) / (64.0 * cntc)[:, None]
    muU = S[:, 320:480].reshape(G, 32, 5).sum(axis=1) / (32.0 * cntc)[:, None]

    w0 = weight[:128]
    w1 = jnp.repeat(weight[128:192], 3)
    w2 = jnp.repeat(weight[192:224], 5)
    alpha = jnp.concatenate([
        inv[:, None] * w0[None, :],
        jnp.broadcast_to(w1[None, :], (G, 192)),
        jnp.broadcast_to(w2[None, :], (G, 160)),
    ], axis=1)
    beta = jnp.concatenate([
        bias[None, :] - (m * inv)[:, None] * w0[None, :],
        -jnp.tile(muT, (1, 64)) * w1[None, :],
        -jnp.tile(muU, (1, 32)) * w2[None, :],
    ], axis=1)

    return _apply_kernel(x, batch, alpha, beta)
